# baseline (speedup 1.0000x reference)
import functools

import jax
import jax.numpy as jnp
from jax import lax
from jax.experimental import pallas as pl
from jax.experimental.pallas import tpu as pltpu
from jax.experimental.pallas import tpu_sc as plsc

N = 10000
E = 320000
F = 128
FH = 64
H1 = 128
H2 = 512
KCHEB = 5
NUM_GRAPHS = 64

NPAD = 10240
TRASH = 10000
NC, NS = 2, 16
CH = 128
NCH = 160
EW = NCH * CH
EPAD = NS * EW
NB = 5
NI = 10
GL = 2
IL = 6
ROWS_PER_TEC = NPAD // NS
_HBM_SET = frozenset({5, 6, 7})

_INTERPRET = False


def _sc_matvec_body(u_hbm, sidx_hbm, didx_hbm, out_hbm,
                    sidx_v, didx_v, buf, usp, acc, gsem, ssem, isg, isd):
    c = lax.axis_index("c")
    s = lax.axis_index("s")

    pltpu.sync_copy(u_hbm.at[c, pl.ds(s * ROWS_PER_TEC, ROWS_PER_TEC)],
                    usp.at[pl.ds(s * ROWS_PER_TEC, ROWS_PER_TEC)])

    def zero_body(i, carry):
        buf[0, i // 4, pl.ds((i % 4) * 16, 16)] = jnp.zeros((16,), jnp.float32)
        return carry
    lax.fori_loop(0, CH * (FH // 16), zero_body, 0)

    def zacc_body(t, carry):
        pltpu.sync_copy(buf.at[0], acc.at[pl.ds(s * ROWS_PER_TEC + t * CH, CH)])
        return carry
    lax.fori_loop(0, ROWS_PER_TEC // CH, zacc_body, 0)

    plsc.subcore_barrier()

    def istage(j, b):
        pltpu.async_copy(sidx_hbm.at[s, j], sidx_v.at[b], isg.at[b])
        pltpu.async_copy(didx_hbm.at[s, j], didx_v.at[b], isd.at[b])

    def iwait(b):
        pltpu.make_async_copy(sidx_hbm.at[s, 0], sidx_v.at[b],
                              isg.at[b]).wait()
        pltpu.make_async_copy(didx_hbm.at[s, 0], didx_v.at[b],
                              isd.at[b]).wait()

    def gather(bi, bd, hbm=False):
        src = u_hbm.at[c].at[sidx_v.at[bi]] if hbm else usp.at[sidx_v.at[bi]]
        pltpu.async_copy(src, buf.at[bd], gsem.at[bd])

    def gwait(b, hbm=False):
        src = u_hbm.at[c].at[sidx_v.at[0]] if hbm else usp.at[sidx_v.at[0]]
        pltpu.make_async_copy(src, buf.at[b], gsem.at[b]).wait()

    def scatter(bi, bd):
        pltpu.async_copy(buf.at[bd], acc.at[didx_v.at[bi]], ssem.at[bd],
                         add=True)

    def swait(b):
        pltpu.make_async_copy(buf.at[b], acc.at[didx_v.at[0]],
                              ssem.at[b]).wait()

    for j in range(IL):
        istage(j, j)
    for j in range(GL):
        iwait(j % NI)
        gather(j % NI, j % NB, hbm=(j % NI) in _HBM_SET)

    def iteration(i, k, first3=False, no_istage=False, no_gather=False):
        if not no_istage:
            istage(i + IL, (k + IL) % NI)
        if not no_gather:
            if not first3:
                swait((k + GL) % NB)
            iwait((k + GL) % NI)
            gather((k + GL) % NI, (k + GL) % NB,
                   hbm=((k + GL) % NI) in _HBM_SET)
        gwait(k % NB, hbm=(k % NI) in _HBM_SET)
        scatter(k % NI, k % NB)

    for i in range(3):
        iteration(i, i, first3=True)

    def main_body(q, carry):
        for t in range(NI):
            iteration(3 + q * NI + t, 3 + t)
        return carry
    lax.fori_loop(0, 15, main_body, 0)

    iteration(153, 153 % NI)
    for i in range(154, 158):
        iteration(i, i % NI, no_istage=True)
    for i in range(158, 160):
        iteration(i, i % NI, no_istage=True, no_gather=True)
    for b in range(NB):
        swait(b)

    plsc.subcore_barrier()

    pltpu.sync_copy(acc.at[pl.ds(s * ROWS_PER_TEC, ROWS_PER_TEC)],
                    out_hbm.at[c, pl.ds(s * ROWS_PER_TEC, ROWS_PER_TEC)])


def _sc_matvec(u2, sidx, didx):
    return pl.kernel(
        _sc_matvec_body,
        out_type=jax.ShapeDtypeStruct((NC, NPAD, FH), jnp.float32),
        mesh=plsc.VectorSubcoreMesh(core_axis_name="c", subcore_axis_name="s",
                                    num_cores=NC, num_subcores=NS),
        scratch_types=[
            pltpu.VMEM((NI, CH), jnp.int32),
            pltpu.VMEM((NI, CH), jnp.int32),
            pltpu.VMEM((NB, CH, FH), jnp.float32),
            pltpu.VMEM_SHARED((NPAD, FH), jnp.float32),
            pltpu.VMEM_SHARED((NPAD, FH), jnp.float32),
            pltpu.SemaphoreType.DMA((NB,)),
            pltpu.SemaphoreType.DMA((NB,)),
            pltpu.SemaphoreType.DMA((NI,)),
            pltpu.SemaphoreType.DMA((NI,)),
        ],
        compiler_params=pltpu.CompilerParams(use_tc_tiling_on_sc=False),
        interpret=_INTERPRET,
    )(u2, sidx, didx)


_BR = 1024


def _halves_to_full(p_ref):
    return jnp.concatenate([p_ref[0], p_ref[1]], axis=1)


def _store_halves(u_ref, u):
    u_ref[0] = u[:, :FH]
    u_ref[1] = u[:, FH:]


def _prep_body(p_ref, x_ref, dinv_ref, u_ref):
    i = pl.program_id(0)
    rows = lax.broadcasted_iota(jnp.int32, (_BR, F), 0) + i * _BR
    deg = _halves_to_full(p_ref)
    valid = (rows < N) & (deg > 0)
    dinv = jnp.where(valid, lax.rsqrt(jnp.maximum(deg, 1e-12)), 0.0)
    dinv_ref[...] = dinv
    _store_halves(u_ref, dinv * x_ref[...])


def _prep(degp, xp):
    return pl.pallas_call(
        _prep_body,
        grid=(NPAD // _BR,),
        in_specs=[
            pl.BlockSpec((NC, _BR, FH), lambda i: (0, i, 0)),
            pl.BlockSpec((_BR, F), lambda i: (i, 0)),
        ],
        out_specs=[
            pl.BlockSpec((_BR, F), lambda i: (i, 0)),
            pl.BlockSpec((NC, _BR, FH), lambda i: (0, i, 0)),
        ],
        out_shape=[
            jax.ShapeDtypeStruct((NPAD, F), jnp.float32),
            jax.ShapeDtypeStruct((NC, NPAD, FH), jnp.float32),
        ],
        interpret=_INTERPRET,
    )(degp, xp)


def _recur_body(p_ref, v_ref, t_ref, d_ref, tx_ref, u_ref, *, ca, cb, cc):
    d = d_ref[...]
    agg = _halves_to_full(p_ref)
    m = ca * (d * agg) + cb * v_ref[...] + cc * t_ref[...]
    tx_ref[...] = m
    _store_halves(u_ref, d * m)


def _recur(p, v, tprev, dinv, ca, cb, cc):
    return pl.pallas_call(
        functools.partial(_recur_body, ca=ca, cb=cb, cc=cc),
        grid=(NPAD // _BR,),
        in_specs=[
            pl.BlockSpec((NC, _BR, FH), lambda i: (0, i, 0)),
            pl.BlockSpec((_BR, F), lambda i: (i, 0)),
            pl.BlockSpec((_BR, F), lambda i: (i, 0)),
            pl.BlockSpec((_BR, F), lambda i: (i, 0)),
        ],
        out_specs=[
            pl.BlockSpec((_BR, F), lambda i: (i, 0)),
            pl.BlockSpec((NC, _BR, FH), lambda i: (0, i, 0)),
        ],
        out_shape=[
            jax.ShapeDtypeStruct((NPAD, F), jnp.float32),
            jax.ShapeDtypeStruct((NC, NPAD, FH), jnp.float32),
        ],
        interpret=_INTERPRET,
    )(p, v, tprev, dinv)


def _scale_body(d_ref, h_ref, u_ref):
    _store_halves(u_ref, d_ref[...] * h_ref[...])


def _scale(dinv, h):
    return pl.pallas_call(
        _scale_body,
        grid=(NPAD // _BR,),
        in_specs=[
            pl.BlockSpec((_BR, F), lambda i: (i, 0)),
            pl.BlockSpec((_BR, F), lambda i: (i, 0)),
        ],
        out_specs=pl.BlockSpec((NC, _BR, FH), lambda i: (0, i, 0)),
        out_shape=jax.ShapeDtypeStruct((NC, NPAD, FH), jnp.float32),
        interpret=_INTERPRET,
    )(dinv, h)


_BM = 512


def _mm_body(*refs, nt, h):
    t_refs = refs[:nt]
    w_ref, b_ref, o_ref = refs[nt], refs[nt + 1], refs[nt + 2]
    acc = jnp.zeros((_BM, h), jnp.float32)
    for j in range(nt):
        acc = acc + jnp.dot(t_refs[j][...], w_ref[pl.ds(j * F, F), :],
                            preferred_element_type=jnp.float32)
    o_ref[...] = jnp.maximum(acc + b_ref[0:1, :], 0.0)


def _mm(ts, wall, bias, h):
    nt = len(ts)
    in_specs = [pl.BlockSpec((_BM, F), lambda i: (i, 0)) for _ in range(nt)]
    in_specs.append(pl.BlockSpec((nt * F, h), lambda i: (0, 0)))
    in_specs.append(pl.BlockSpec((8, h), lambda i: (0, 0)))
    return pl.pallas_call(
        functools.partial(_mm_body, nt=nt, h=h),
        grid=(NPAD // _BM,),
        in_specs=in_specs,
        out_specs=pl.BlockSpec((_BM, h), lambda i: (i, 0)),
        out_shape=jax.ShapeDtypeStruct((NPAD, h), jnp.float32),
        interpret=_INTERPRET,
    )(*ts, wall, bias)


_CR = 1024


def _pool_body(h_ref, b_ref, o_ref, acc_ref, cnt_ref):
    i = pl.program_id(0)

    @pl.when(i == 0)
    def _():
        acc_ref[...] = jnp.zeros_like(acc_ref)
        cnt_ref[...] = jnp.zeros_like(cnt_ref)

    b = b_ref[0]
    gids = lax.broadcasted_iota(jnp.int32, (NUM_GRAPHS, _CR), 0)
    rows = lax.broadcasted_iota(jnp.int32, (NUM_GRAPHS, _CR), 1) + i * _CR
    p = jnp.where((b == gids) & (rows < N), 1.0, 0.0)
    acc_ref[...] += jnp.dot(p, h_ref[...], preferred_element_type=jnp.float32)
    cnt_ref[...] += jnp.broadcast_to(jnp.sum(p, axis=1, keepdims=True),
                                     (NUM_GRAPHS, 128))

    @pl.when(i == NPAD // _CR - 1)
    def _():
        cnt = jnp.maximum(cnt_ref[...][:, 0:1], 1.0)
        o_ref[...] = acc_ref[...] / cnt


def _pool(h2, batch3d):
    return pl.pallas_call(
        _pool_body,
        grid=(NPAD // _CR,),
        in_specs=[
            pl.BlockSpec((_CR, H2), lambda i: (i, 0)),
            pl.BlockSpec((1, 1, _CR), lambda i: (i, 0, 0)),
        ],
        out_specs=pl.BlockSpec((NUM_GRAPHS, H2), lambda i: (0, 0)),
        out_shape=jax.ShapeDtypeStruct((NUM_GRAPHS, H2), jnp.float32),
        scratch_shapes=[
            pltpu.VMEM((NUM_GRAPHS, H2), jnp.float32),
            pltpu.VMEM((NUM_GRAPHS, 128), jnp.float32),
        ],
        interpret=_INTERPRET,
    )(h2, batch3d)


def _cheb_txs(xp, dinv, u0, colp, rowp):
    txs = [xp]
    u_cur = u0
    for k in range(1, KCHEB):
        p = _sc_matvec(u_cur, colp, rowp)
        if k == 1:
            tx, u_cur = _recur(p, xp, xp, dinv, -2.0 / 3.0, -1.0 / 3.0, 0.0)
        else:
            tx, u_cur = _recur(p, txs[-1], txs[-2], dinv,
                               -4.0 / 3.0, -2.0 / 3.0, -1.0)
        txs.append(tx)
    return txs


def kernel(x, edge_index, batch, W1f, b1f, W1b, b1b, W2f, b2f, W2b, b2b):
    f32 = jnp.float32
    row = edge_index[0]
    col = edge_index[1]
    pad = jnp.full((EPAD - E,), TRASH, jnp.int32)
    rowp = jnp.concatenate([row, pad]).reshape(NS, NCH, CH)
    colp = jnp.concatenate([col, pad]).reshape(NS, NCH, CH)

    xp = jnp.zeros((NPAD, F), f32).at[:N].set(x)
    ones2 = jnp.zeros((NC, NPAD, FH), f32).at[:, :N].set(1.0)
    batch3d = jnp.zeros((NPAD,), jnp.int32).at[:N].set(batch) \
        .reshape(NPAD // _CR, 1, _CR)

    degp = _sc_matvec(ones2, colp, rowp)
    dinv, u0 = _prep(degp, xp)

    txs_f = _cheb_txs(xp, dinv, u0, colp, rowp)
    txs_b = _cheb_txs(xp, dinv, u0, rowp, colp)
    w1 = jnp.concatenate([W1f.reshape(KCHEB * F, H1),
                          W1b.reshape(KCHEB * F, H1)], axis=0)
    bias1 = jnp.tile((b1f + b1b)[None, :], (8, 1))
    h = _mm(txs_f + txs_b, w1, bias1, H1)

    uh = _scale(dinv, h)
    txs_f2 = _cheb_txs(h, dinv, uh, colp, rowp)
    txs_b2 = _cheb_txs(h, dinv, uh, rowp, colp)
    w2 = jnp.concatenate([W2f.reshape(KCHEB * H1, H2),
                          W2b.reshape(KCHEB * H1, H2)], axis=0)
    bias2 = jnp.tile((b2f + b2b)[None, :], (8, 1))
    h2 = _mm(txs_f2 + txs_b2, w2, bias2, H2)

    return _pool(h2, batch3d)


# speedup vs baseline: 1.4120x; 1.4120x over previous
import functools

import jax
import jax.numpy as jnp
from jax import lax
from jax.experimental import pallas as pl
from jax.experimental.pallas import tpu as pltpu
from jax.experimental.pallas import tpu_sc as plsc

N = 10000
E = 320000
F = 128
FH = 64
H1 = 128
H2 = 512
KCHEB = 5
NUM_GRAPHS = 64

NPAD = 10240
TRASH = 10000
NC, NS = 2, 16
CH = 128
NCH = 160
EW = NCH * CH
EPAD = NS * EW
NB = 5
NI = 10
GL = 2
IL = 6
ROWS_PER_TEC = NPAD // NS
_HBM_SET = frozenset()

_INTERPRET = False


def _sc_matvec_body(u_hbm, sidx_hbm, didx_hbm, out_hbm,
                    sidx_v, didx_v, buf, usp, acc, gsem, ssem, isg, isd):
    c = lax.axis_index("c")
    s = lax.axis_index("s")

    pltpu.sync_copy(u_hbm.at[c, pl.ds(s * ROWS_PER_TEC, ROWS_PER_TEC)],
                    usp.at[pl.ds(s * ROWS_PER_TEC, ROWS_PER_TEC)])

    def zero_body(i, carry):
        buf[0, i // 4, pl.ds((i % 4) * 16, 16)] = jnp.zeros((16,), jnp.float32)
        return carry
    lax.fori_loop(0, CH * (FH // 16), zero_body, 0)

    def zacc_body(t, carry):
        pltpu.sync_copy(buf.at[0], acc.at[pl.ds(s * ROWS_PER_TEC + t * CH, CH)])
        return carry
    lax.fori_loop(0, ROWS_PER_TEC // CH, zacc_body, 0)

    plsc.subcore_barrier()

    def istage(j, b):
        pltpu.async_copy(sidx_hbm.at[s, j], sidx_v.at[b], isg.at[b])
        pltpu.async_copy(didx_hbm.at[s, j], didx_v.at[b], isd.at[b])

    def iwait(b):
        pltpu.make_async_copy(sidx_hbm.at[s, 0], sidx_v.at[b],
                              isg.at[b]).wait()
        pltpu.make_async_copy(didx_hbm.at[s, 0], didx_v.at[b],
                              isd.at[b]).wait()

    def gather(bi, bd, hbm=False):
        src = u_hbm.at[c].at[sidx_v.at[bi]] if hbm else usp.at[sidx_v.at[bi]]
        pltpu.async_copy(src, buf.at[bd], gsem.at[bd])

    def gwait(b, hbm=False):
        src = u_hbm.at[c].at[sidx_v.at[0]] if hbm else usp.at[sidx_v.at[0]]
        pltpu.make_async_copy(src, buf.at[b], gsem.at[b]).wait()

    def scatter(bi, bd):
        pltpu.async_copy(buf.at[bd], acc.at[didx_v.at[bi]], ssem.at[bd],
                         add=True)

    def swait(b):
        pltpu.make_async_copy(buf.at[b], acc.at[didx_v.at[0]],
                              ssem.at[b]).wait()

    for j in range(IL):
        istage(j, j)
    for j in range(GL):
        iwait(j % NI)
        gather(j % NI, j % NB, hbm=(j % NI) in _HBM_SET)

    def iteration(i, k, first3=False, no_istage=False, no_gather=False):
        if not no_istage:
            istage(i + IL, (k + IL) % NI)
        if not no_gather:
            if not first3:
                swait((k + GL) % NB)
            iwait((k + GL) % NI)
            gather((k + GL) % NI, (k + GL) % NB,
                   hbm=((k + GL) % NI) in _HBM_SET)
        gwait(k % NB, hbm=(k % NI) in _HBM_SET)
        scatter(k % NI, k % NB)

    for i in range(3):
        iteration(i, i, first3=True)

    def main_body(q, carry):
        for t in range(NI):
            iteration(3 + q * NI + t, 3 + t)
        return carry
    lax.fori_loop(0, 15, main_body, 0)

    iteration(153, 153 % NI)
    for i in range(154, 158):
        iteration(i, i % NI, no_istage=True)
    for i in range(158, 160):
        iteration(i, i % NI, no_istage=True, no_gather=True)
    for b in range(NB):
        swait(b)

    plsc.subcore_barrier()

    pltpu.sync_copy(acc.at[pl.ds(s * ROWS_PER_TEC, ROWS_PER_TEC)],
                    out_hbm.at[c, pl.ds(s * ROWS_PER_TEC, ROWS_PER_TEC)])


def _sc_degree_body(didx_hbm, out_hbm, didx_v, buf, acc, ssem, isd):
    c = lax.axis_index("c")
    s = lax.axis_index("s")
    NCHD = NCH // NC

    def fill_body(i, carry):
        buf[0, i // 4, pl.ds((i % 4) * 16, 16)] = jnp.zeros((16,), jnp.float32)
        buf[1, i // 4, pl.ds((i % 4) * 16, 16)] = jnp.ones((16,), jnp.float32)
        return carry
    lax.fori_loop(0, CH * (FH // 16), fill_body, 0)

    def zacc_body(t, carry):
        pltpu.sync_copy(buf.at[0], acc.at[pl.ds(s * ROWS_PER_TEC + t * CH, CH)])
        return carry
    lax.fori_loop(0, ROWS_PER_TEC // CH, zacc_body, 0)

    plsc.subcore_barrier()

    ILD = 4

    def istage(j, b):
        pltpu.async_copy(didx_hbm.at[s, c * NCHD + j], didx_v.at[b],
                         isd.at[b])

    def iwait(b):
        pltpu.make_async_copy(didx_hbm.at[s, 0], didx_v.at[b],
                              isd.at[b]).wait()

    def scatter(bi, bd):
        pltpu.async_copy(buf.at[1], acc.at[didx_v.at[bi]], ssem.at[bd],
                         add=True)

    def swait(b):
        pltpu.make_async_copy(buf.at[1], acc.at[didx_v.at[0]],
                              ssem.at[b]).wait()

    for j in range(ILD):
        istage(j, j)

    def iteration(i, k, warm, no_istage=False):
        if not no_istage:
            istage(i + ILD, (k + ILD) % NI)
        iwait(k % NI)
        if warm:
            swait(k % NB)
        scatter(k % NI, k % NB)

    for i in range(NB):
        iteration(i, i, warm=False)

    def main_body(q, carry):
        for t in range(NI):
            iteration(NB + q * NI + t, NB + t, warm=True)
        return carry
    lax.fori_loop(0, (NCHD - NB - 15) // NI, main_body, 0)

    for i in range(NCHD - 15, NCHD):
        iteration(i, i % NI, warm=True, no_istage=(i + ILD >= NCHD))
    for b in range(NB):
        swait(b)

    plsc.subcore_barrier()
    pltpu.sync_copy(acc.at[pl.ds(s * ROWS_PER_TEC, ROWS_PER_TEC)],
                    out_hbm.at[c, pl.ds(s * ROWS_PER_TEC, ROWS_PER_TEC)])


def _sc_degree(didx):
    return pl.kernel(
        _sc_degree_body,
        out_type=jax.ShapeDtypeStruct((NC, NPAD, FH), jnp.float32),
        mesh=plsc.VectorSubcoreMesh(core_axis_name="c", subcore_axis_name="s",
                                    num_cores=NC, num_subcores=NS),
        scratch_types=[
            pltpu.VMEM((NI, CH), jnp.int32),
            pltpu.VMEM((2, CH, FH), jnp.float32),
            pltpu.VMEM_SHARED((NPAD, FH), jnp.float32),
            pltpu.SemaphoreType.DMA((NB,)),
            pltpu.SemaphoreType.DMA((NI,)),
        ],
        compiler_params=pltpu.CompilerParams(use_tc_tiling_on_sc=False),
        interpret=_INTERPRET,
    )(didx)


def _sc_matvec(u2, sidx, didx):
    return pl.kernel(
        _sc_matvec_body,
        out_type=jax.ShapeDtypeStruct((NC, NPAD, FH), jnp.float32),
        mesh=plsc.VectorSubcoreMesh(core_axis_name="c", subcore_axis_name="s",
                                    num_cores=NC, num_subcores=NS),
        scratch_types=[
            pltpu.VMEM((NI, CH), jnp.int32),
            pltpu.VMEM((NI, CH), jnp.int32),
            pltpu.VMEM((NB, CH, FH), jnp.float32),
            pltpu.VMEM_SHARED((NPAD, FH), jnp.float32),
            pltpu.VMEM_SHARED((NPAD, FH), jnp.float32),
            pltpu.SemaphoreType.DMA((NB,)),
            pltpu.SemaphoreType.DMA((NB,)),
            pltpu.SemaphoreType.DMA((NI,)),
            pltpu.SemaphoreType.DMA((NI,)),
        ],
        compiler_params=pltpu.CompilerParams(use_tc_tiling_on_sc=False),
        interpret=_INTERPRET,
    )(u2, sidx, didx)


_BR = 1024


def _halves_to_full(p_ref):
    return jnp.concatenate([p_ref[0], p_ref[1]], axis=1)


def _store_halves(u_ref, u):
    u_ref[0] = u[:, :FH]
    u_ref[1] = u[:, FH:]


def _prep_body(p_ref, x_ref, dinv_ref, u_ref):
    i = pl.program_id(0)
    rows = lax.broadcasted_iota(jnp.int32, (_BR, F), 0) + i * _BR
    deg_h = p_ref[0] + p_ref[1]
    deg = jnp.concatenate([deg_h, deg_h], axis=1)
    valid = (rows < N) & (deg > 0)
    dinv = jnp.where(valid, lax.rsqrt(jnp.maximum(deg, 1e-12)), 0.0)
    dinv_ref[...] = dinv
    _store_halves(u_ref, dinv * x_ref[...])


def _prep(degp, xp):
    return pl.pallas_call(
        _prep_body,
        grid=(NPAD // _BR,),
        in_specs=[
            pl.BlockSpec((NC, _BR, FH), lambda i: (0, i, 0)),
            pl.BlockSpec((_BR, F), lambda i: (i, 0)),
        ],
        out_specs=[
            pl.BlockSpec((_BR, F), lambda i: (i, 0)),
            pl.BlockSpec((NC, _BR, FH), lambda i: (0, i, 0)),
        ],
        out_shape=[
            jax.ShapeDtypeStruct((NPAD, F), jnp.float32),
            jax.ShapeDtypeStruct((NC, NPAD, FH), jnp.float32),
        ],
        interpret=_INTERPRET,
    )(degp, xp)


def _recur_body(p_ref, v_ref, t_ref, d_ref, tx_ref, u_ref, *, ca, cb, cc):
    d = d_ref[...]
    agg = _halves_to_full(p_ref)
    m = ca * (d * agg) + cb * v_ref[...] + cc * t_ref[...]
    tx_ref[...] = m
    _store_halves(u_ref, d * m)


def _recur(p, v, tprev, dinv, ca, cb, cc):
    return pl.pallas_call(
        functools.partial(_recur_body, ca=ca, cb=cb, cc=cc),
        grid=(NPAD // _BR,),
        in_specs=[
            pl.BlockSpec((NC, _BR, FH), lambda i: (0, i, 0)),
            pl.BlockSpec((_BR, F), lambda i: (i, 0)),
            pl.BlockSpec((_BR, F), lambda i: (i, 0)),
            pl.BlockSpec((_BR, F), lambda i: (i, 0)),
        ],
        out_specs=[
            pl.BlockSpec((_BR, F), lambda i: (i, 0)),
            pl.BlockSpec((NC, _BR, FH), lambda i: (0, i, 0)),
        ],
        out_shape=[
            jax.ShapeDtypeStruct((NPAD, F), jnp.float32),
            jax.ShapeDtypeStruct((NC, NPAD, FH), jnp.float32),
        ],
        interpret=_INTERPRET,
    )(p, v, tprev, dinv)


def _scale_body(d_ref, h_ref, u_ref):
    _store_halves(u_ref, d_ref[...] * h_ref[...])


def _scale(dinv, h):
    return pl.pallas_call(
        _scale_body,
        grid=(NPAD // _BR,),
        in_specs=[
            pl.BlockSpec((_BR, F), lambda i: (i, 0)),
            pl.BlockSpec((_BR, F), lambda i: (i, 0)),
        ],
        out_specs=pl.BlockSpec((NC, _BR, FH), lambda i: (0, i, 0)),
        out_shape=jax.ShapeDtypeStruct((NC, NPAD, FH), jnp.float32),
        interpret=_INTERPRET,
    )(dinv, h)


_BM = 512


def _mm_body(*refs, nt, h):
    t_refs = refs[:nt]
    w_ref, b_ref, o_ref = refs[nt], refs[nt + 1], refs[nt + 2]
    acc = jnp.zeros((_BM, h), jnp.float32)
    for j in range(nt):
        acc = acc + jnp.dot(t_refs[j][...], w_ref[pl.ds(j * F, F), :],
                            preferred_element_type=jnp.float32)
    o_ref[...] = jnp.maximum(acc + b_ref[0:1, :], 0.0)


def _mm(ts, wall, bias, h):
    nt = len(ts)
    in_specs = [pl.BlockSpec((_BM, F), lambda i: (i, 0)) for _ in range(nt)]
    in_specs.append(pl.BlockSpec((nt * F, h), lambda i: (0, 0)))
    in_specs.append(pl.BlockSpec((8, h), lambda i: (0, 0)))
    return pl.pallas_call(
        functools.partial(_mm_body, nt=nt, h=h),
        grid=(NPAD // _BM,),
        in_specs=in_specs,
        out_specs=pl.BlockSpec((_BM, h), lambda i: (i, 0)),
        out_shape=jax.ShapeDtypeStruct((NPAD, h), jnp.float32),
        interpret=_INTERPRET,
    )(*ts, wall, bias)


_CR = 1024


def _pool_body(h_ref, b_ref, o_ref, acc_ref, cnt_ref):
    i = pl.program_id(0)

    @pl.when(i == 0)
    def _():
        acc_ref[...] = jnp.zeros_like(acc_ref)
        cnt_ref[...] = jnp.zeros_like(cnt_ref)

    b = b_ref[0]
    gids = lax.broadcasted_iota(jnp.int32, (NUM_GRAPHS, _CR), 0)
    rows = lax.broadcasted_iota(jnp.int32, (NUM_GRAPHS, _CR), 1) + i * _CR
    p = jnp.where((b == gids) & (rows < N), 1.0, 0.0)
    acc_ref[...] += jnp.dot(p, h_ref[...], preferred_element_type=jnp.float32)
    cnt_ref[...] += jnp.broadcast_to(jnp.sum(p, axis=1, keepdims=True),
                                     (NUM_GRAPHS, 128))

    @pl.when(i == NPAD // _CR - 1)
    def _():
        cnt = jnp.maximum(cnt_ref[...][:, 0:1], 1.0)
        o_ref[...] = acc_ref[...] / cnt


def _pool(h2, batch3d):
    return pl.pallas_call(
        _pool_body,
        grid=(NPAD // _CR,),
        in_specs=[
            pl.BlockSpec((_CR, H2), lambda i: (i, 0)),
            pl.BlockSpec((1, 1, _CR), lambda i: (i, 0, 0)),
        ],
        out_specs=pl.BlockSpec((NUM_GRAPHS, H2), lambda i: (0, 0)),
        out_shape=jax.ShapeDtypeStruct((NUM_GRAPHS, H2), jnp.float32),
        scratch_shapes=[
            pltpu.VMEM((NUM_GRAPHS, H2), jnp.float32),
            pltpu.VMEM((NUM_GRAPHS, 128), jnp.float32),
        ],
        interpret=_INTERPRET,
    )(h2, batch3d)


def _cheb_pair(v0, dinv, u0, colp, rowp):
    txf, txb = [v0], [v0]
    uf = ub = u0
    for k in range(1, KCHEB):
        pf = _sc_matvec(uf, colp, rowp)
        pb = _sc_matvec(ub, rowp, colp)
        if k == 1:
            tf, uf = _recur(pf, v0, v0, dinv, -2.0 / 3.0, -1.0 / 3.0, 0.0)
            tb, ub = _recur(pb, v0, v0, dinv, -2.0 / 3.0, -1.0 / 3.0, 0.0)
        else:
            tf, uf = _recur(pf, txf[-1], txf[-2], dinv,
                            -4.0 / 3.0, -2.0 / 3.0, -1.0)
            tb, ub = _recur(pb, txb[-1], txb[-2], dinv,
                            -4.0 / 3.0, -2.0 / 3.0, -1.0)
        txf.append(tf)
        txb.append(tb)
    return txf, txb


def kernel(x, edge_index, batch, W1f, b1f, W1b, b1b, W2f, b2f, W2b, b2b):
    f32 = jnp.float32
    row = edge_index[0]
    col = edge_index[1]
    pad = jnp.full((EPAD - E,), TRASH, jnp.int32)
    rowp = jnp.concatenate([row, pad]).reshape(NS, NCH, CH)
    colp = jnp.concatenate([col, pad]).reshape(NS, NCH, CH)

    xp = jnp.zeros((NPAD, F), f32).at[:N].set(x)
    batch3d = jnp.zeros((NPAD,), jnp.int32).at[:N].set(batch) \
        .reshape(NPAD // _CR, 1, _CR)

    degp = _sc_degree(rowp)
    dinv, u0 = _prep(degp, xp)

    txs_f, txs_b = _cheb_pair(xp, dinv, u0, colp, rowp)
    w1 = jnp.concatenate([W1f.reshape(KCHEB * F, H1),
                          W1b.reshape(KCHEB * F, H1)], axis=0)
    bias1 = jnp.tile((b1f + b1b)[None, :], (8, 1))
    h = _mm(txs_f + txs_b, w1, bias1, H1)

    uh = _scale(dinv, h)
    txs_f2, txs_b2 = _cheb_pair(h, dinv, uh, colp, rowp)
    w2 = jnp.concatenate([W2f.reshape(KCHEB * H1, H2),
                          W2b.reshape(KCHEB * H1, H2)], axis=0)
    bias2 = jnp.tile((b2f + b2b)[None, :], (8, 1))
    h2 = _mm(txs_f2 + txs_b2, w2, bias2, H2)

    return _pool(h2, batch3d)


# speedup vs baseline: 1.4204x; 1.0060x over previous
import functools

import jax
import jax.numpy as jnp
from jax import lax
from jax.experimental import pallas as pl
from jax.experimental.pallas import tpu as pltpu
from jax.experimental.pallas import tpu_sc as plsc

N = 10000
E = 320000
F = 128
FH = 64
H1 = 128
H2 = 512
KCHEB = 5
NUM_GRAPHS = 64

NPAD = 10240
TRASH = 10000
NC, NS = 2, 16
CH = 128
NCH = 160
EW = NCH * CH
EPAD = NS * EW
NB = 5
NI = 10
GL = 2
IL = 6
ROWS_PER_TEC = NPAD // NS
_HBM_SET = frozenset()

_INTERPRET = False


def _sc_matvec_body(u_hbm, sidx_hbm, didx_hbm, out_hbm,
                    sidx_v, didx_v, buf, usp, acc, gsem, ssem, isg, isd):
    c = lax.axis_index("c")
    s = lax.axis_index("s")

    pltpu.sync_copy(u_hbm.at[c, pl.ds(s * ROWS_PER_TEC, ROWS_PER_TEC)],
                    usp.at[pl.ds(s * ROWS_PER_TEC, ROWS_PER_TEC)])

    def zero_body(i, carry):
        buf[0, i // 4, pl.ds((i % 4) * 16, 16)] = jnp.zeros((16,), jnp.float32)
        return carry
    lax.fori_loop(0, CH * (FH // 16), zero_body, 0)

    def zacc_body(t, carry):
        pltpu.sync_copy(buf.at[0], acc.at[pl.ds(s * ROWS_PER_TEC + t * CH, CH)])
        return carry
    lax.fori_loop(0, ROWS_PER_TEC // CH, zacc_body, 0)

    plsc.subcore_barrier()

    def istage(j, b):
        pltpu.async_copy(sidx_hbm.at[s, j], sidx_v.at[b], isg.at[b])
        pltpu.async_copy(didx_hbm.at[s, j], didx_v.at[b], isd.at[b])

    def iwait(b):
        pltpu.make_async_copy(sidx_hbm.at[s, 0], sidx_v.at[b],
                              isg.at[b]).wait()
        pltpu.make_async_copy(didx_hbm.at[s, 0], didx_v.at[b],
                              isd.at[b]).wait()

    def gather(bi, bd, hbm=False):
        src = u_hbm.at[c].at[sidx_v.at[bi]] if hbm else usp.at[sidx_v.at[bi]]
        pltpu.async_copy(src, buf.at[bd], gsem.at[bd])

    def gwait(b, hbm=False):
        src = u_hbm.at[c].at[sidx_v.at[0]] if hbm else usp.at[sidx_v.at[0]]
        pltpu.make_async_copy(src, buf.at[b], gsem.at[b]).wait()

    def scatter(bi, bd):
        pltpu.async_copy(buf.at[bd], acc.at[didx_v.at[bi]], ssem.at[bd],
                         add=True)

    def swait(b):
        pltpu.make_async_copy(buf.at[b], acc.at[didx_v.at[0]],
                              ssem.at[b]).wait()

    for j in range(IL):
        istage(j, j)
    for j in range(GL):
        iwait(j % NI)
        gather(j % NI, j % NB, hbm=(j % NI) in _HBM_SET)

    def iteration(i, k, first3=False, no_istage=False, no_gather=False):
        if not no_istage:
            istage(i + IL, (k + IL) % NI)
        if not no_gather:
            if not first3:
                swait((k + GL) % NB)
            iwait((k + GL) % NI)
            gather((k + GL) % NI, (k + GL) % NB,
                   hbm=((k + GL) % NI) in _HBM_SET)
        gwait(k % NB, hbm=(k % NI) in _HBM_SET)
        scatter(k % NI, k % NB)

    for i in range(3):
        iteration(i, i, first3=True)

    def main_body(q, carry):
        for t in range(NI):
            iteration(3 + q * NI + t, 3 + t)
        return carry
    lax.fori_loop(0, 15, main_body, 0)

    iteration(153, 153 % NI)
    for i in range(154, 158):
        iteration(i, i % NI, no_istage=True)
    for i in range(158, 160):
        iteration(i, i % NI, no_istage=True, no_gather=True)
    for b in range(NB):
        swait(b)

    plsc.subcore_barrier()

    pltpu.sync_copy(acc.at[pl.ds(s * ROWS_PER_TEC, ROWS_PER_TEC)],
                    out_hbm.at[c, pl.ds(s * ROWS_PER_TEC, ROWS_PER_TEC)])


def _sc_degree_body(didx_hbm, out_hbm, didx_v, buf, acc, ssem, isd):
    c = lax.axis_index("c")
    s = lax.axis_index("s")
    NCHD = NCH // NC

    def fill_body(i, carry):
        buf[0, i // 4, pl.ds((i % 4) * 16, 16)] = jnp.zeros((16,), jnp.float32)
        buf[1, i // 4, pl.ds((i % 4) * 16, 16)] = jnp.ones((16,), jnp.float32)
        return carry
    lax.fori_loop(0, CH * (FH // 16), fill_body, 0)

    def zacc_body(t, carry):
        pltpu.sync_copy(buf.at[0], acc.at[pl.ds(s * ROWS_PER_TEC + t * CH, CH)])
        return carry
    lax.fori_loop(0, ROWS_PER_TEC // CH, zacc_body, 0)

    plsc.subcore_barrier()

    ILD = 4

    def istage(j, b):
        pltpu.async_copy(didx_hbm.at[s, c * NCHD + j], didx_v.at[b],
                         isd.at[b])

    def iwait(b):
        pltpu.make_async_copy(didx_hbm.at[s, 0], didx_v.at[b],
                              isd.at[b]).wait()

    def scatter(bi, bd):
        pltpu.async_copy(buf.at[1], acc.at[didx_v.at[bi]], ssem.at[bd],
                         add=True)

    def swait(b):
        pltpu.make_async_copy(buf.at[1], acc.at[didx_v.at[0]],
                              ssem.at[b]).wait()

    for j in range(ILD):
        istage(j, j)

    def iteration(i, k, warm, no_istage=False):
        if not no_istage:
            istage(i + ILD, (k + ILD) % NI)
        iwait(k % NI)
        if warm:
            swait(k % NB)
        scatter(k % NI, k % NB)

    for i in range(NB):
        iteration(i, i, warm=False)

    def main_body(q, carry):
        for t in range(NI):
            iteration(NB + q * NI + t, NB + t, warm=True)
        return carry
    lax.fori_loop(0, (NCHD - NB - 15) // NI, main_body, 0)

    for i in range(NCHD - 15, NCHD):
        iteration(i, i % NI, warm=True, no_istage=(i + ILD >= NCHD))
    for b in range(NB):
        swait(b)

    plsc.subcore_barrier()
    pltpu.sync_copy(acc.at[pl.ds(s * ROWS_PER_TEC, ROWS_PER_TEC)],
                    out_hbm.at[c, pl.ds(s * ROWS_PER_TEC, ROWS_PER_TEC)])


def _sc_degree(didx):
    return pl.kernel(
        _sc_degree_body,
        out_type=jax.ShapeDtypeStruct((NC, NPAD, FH), jnp.float32),
        mesh=plsc.VectorSubcoreMesh(core_axis_name="c", subcore_axis_name="s",
                                    num_cores=NC, num_subcores=NS),
        scratch_types=[
            pltpu.VMEM((NI, CH), jnp.int32),
            pltpu.VMEM((2, CH, FH), jnp.float32),
            pltpu.VMEM_SHARED((NPAD, FH), jnp.float32),
            pltpu.SemaphoreType.DMA((NB,)),
            pltpu.SemaphoreType.DMA((NI,)),
        ],
        compiler_params=pltpu.CompilerParams(use_tc_tiling_on_sc=False),
        interpret=_INTERPRET,
    )(didx)


def _sc_matvec(u2, sidx, didx):
    return pl.kernel(
        _sc_matvec_body,
        out_type=jax.ShapeDtypeStruct((NC, NPAD, FH), jnp.float32),
        mesh=plsc.VectorSubcoreMesh(core_axis_name="c", subcore_axis_name="s",
                                    num_cores=NC, num_subcores=NS),
        scratch_types=[
            pltpu.VMEM((NI, CH), jnp.int32),
            pltpu.VMEM((NI, CH), jnp.int32),
            pltpu.VMEM((NB, CH, FH), jnp.float32),
            pltpu.VMEM_SHARED((NPAD, FH), jnp.float32),
            pltpu.VMEM_SHARED((NPAD, FH), jnp.float32),
            pltpu.SemaphoreType.DMA((NB,)),
            pltpu.SemaphoreType.DMA((NB,)),
            pltpu.SemaphoreType.DMA((NI,)),
            pltpu.SemaphoreType.DMA((NI,)),
        ],
        compiler_params=pltpu.CompilerParams(use_tc_tiling_on_sc=False),
        interpret=_INTERPRET,
    )(u2, sidx, didx)


_BR = 1024


def _halves_to_full(p_ref):
    return jnp.concatenate([p_ref[0], p_ref[1]], axis=1)


def _store_halves(u_ref, u):
    u_ref[0] = u[:, :FH]
    u_ref[1] = u[:, FH:]


def _prep_body(p_ref, x_ref, dinv_ref, u_ref):
    i = pl.program_id(0)
    rows = lax.broadcasted_iota(jnp.int32, (_BR, F), 0) + i * _BR
    deg_h = p_ref[0] + p_ref[1]
    deg = jnp.concatenate([deg_h, deg_h], axis=1)
    valid = (rows < N) & (deg > 0)
    dinv = jnp.where(valid, lax.rsqrt(jnp.maximum(deg, 1e-12)), 0.0)
    dinv_ref[...] = dinv
    _store_halves(u_ref, dinv * x_ref[...])


def _prep(degp, xp):
    return pl.pallas_call(
        _prep_body,
        grid=(NPAD // _BR,),
        in_specs=[
            pl.BlockSpec((NC, _BR, FH), lambda i: (0, i, 0)),
            pl.BlockSpec((_BR, F), lambda i: (i, 0)),
        ],
        out_specs=[
            pl.BlockSpec((_BR, F), lambda i: (i, 0)),
            pl.BlockSpec((NC, _BR, FH), lambda i: (0, i, 0)),
        ],
        out_shape=[
            jax.ShapeDtypeStruct((NPAD, F), jnp.float32),
            jax.ShapeDtypeStruct((NC, NPAD, FH), jnp.float32),
        ],
        interpret=_INTERPRET,
    )(degp, xp)


def _recur_body(p_ref, v_ref, t_ref, d_ref, tx_ref, u_ref, *, ca, cb, cc):
    d = d_ref[...]
    m = ca * (d * _halves_to_full(p_ref)) + cb * v_ref[...] + cc * t_ref[...]
    tx_ref[...] = m
    _store_halves(u_ref, d * m)


def _recur(p, v, tprev, dinv, ca, cb, cc):
    nspec = pl.BlockSpec((_BR, F), lambda i: (i, 0))
    hspec = pl.BlockSpec((NC, _BR, FH), lambda i: (0, i, 0))
    return pl.pallas_call(
        functools.partial(_recur_body, ca=ca, cb=cb, cc=cc),
        grid=(NPAD // _BR,),
        in_specs=[hspec, nspec, nspec, nspec],
        out_specs=[nspec, hspec],
        out_shape=[
            jax.ShapeDtypeStruct((NPAD, F), jnp.float32),
            jax.ShapeDtypeStruct((NC, NPAD, FH), jnp.float32),
        ],
        interpret=_INTERPRET,
    )(p, v, tprev, dinv)


def _scale_body(d_ref, h_ref, u_ref):
    _store_halves(u_ref, d_ref[...] * h_ref[...])


def _scale(dinv, h):
    return pl.pallas_call(
        _scale_body,
        grid=(NPAD // _BR,),
        in_specs=[
            pl.BlockSpec((_BR, F), lambda i: (i, 0)),
            pl.BlockSpec((_BR, F), lambda i: (i, 0)),
        ],
        out_specs=pl.BlockSpec((NC, _BR, FH), lambda i: (0, i, 0)),
        out_shape=jax.ShapeDtypeStruct((NC, NPAD, FH), jnp.float32),
        interpret=_INTERPRET,
    )(dinv, h)


_BM = 512


def _mm_body(*refs, nt, h):
    t_refs = refs[:nt]
    w_ref, b_ref, o_ref = refs[nt], refs[nt + 1], refs[nt + 2]
    acc = jnp.zeros((_BM, h), jnp.float32)
    for j in range(nt):
        acc = acc + jnp.dot(t_refs[j][...], w_ref[pl.ds(j * F, F), :],
                            preferred_element_type=jnp.float32)
    o_ref[...] = jnp.maximum(acc + b_ref[0:1, :], 0.0)


def _mm(ts, wall, bias, h):
    nt = len(ts)
    in_specs = [pl.BlockSpec((_BM, F), lambda i: (i, 0)) for _ in range(nt)]
    in_specs.append(pl.BlockSpec((nt * F, h), lambda i: (0, 0)))
    in_specs.append(pl.BlockSpec((8, h), lambda i: (0, 0)))
    return pl.pallas_call(
        functools.partial(_mm_body, nt=nt, h=h),
        grid=(NPAD // _BM,),
        in_specs=in_specs,
        out_specs=pl.BlockSpec((_BM, h), lambda i: (i, 0)),
        out_shape=jax.ShapeDtypeStruct((NPAD, h), jnp.float32),
        interpret=_INTERPRET,
    )(*ts, wall, bias)


def _mm_pool_body(*refs, nt):
    t_refs = refs[:nt]
    w_ref, b_ref, bat_ref, o_ref, acc_ref, cnt_ref = refs[nt:nt + 6]
    i = pl.program_id(0)
    acc = jnp.zeros((_BM, H2), jnp.float32)
    for j in range(nt):
        acc = acc + jnp.dot(t_refs[j][...], w_ref[pl.ds(j * F, F), :],
                            preferred_element_type=jnp.float32)
    hblk = jnp.maximum(acc + b_ref[0:1, :], 0.0)

    @pl.when(i == 0)
    def _():
        acc_ref[...] = jnp.zeros_like(acc_ref)
        cnt_ref[...] = jnp.zeros_like(cnt_ref)

    bat = bat_ref[0]
    gids = lax.broadcasted_iota(jnp.int32, (NUM_GRAPHS, _BM), 0)
    rows = lax.broadcasted_iota(jnp.int32, (NUM_GRAPHS, _BM), 1) + i * _BM
    p = jnp.where((bat == gids) & (rows < N), 1.0, 0.0)
    acc_ref[...] += jnp.dot(p, hblk, preferred_element_type=jnp.float32)
    cnt_ref[...] += jnp.broadcast_to(jnp.sum(p, axis=1, keepdims=True),
                                     (NUM_GRAPHS, 128))

    @pl.when(i == NPAD // _BM - 1)
    def _():
        cnt = jnp.maximum(cnt_ref[...][:, 0:1], 1.0)
        o_ref[...] = acc_ref[...] / cnt


def _mm_pool(ts, wall, bias, batch3d):
    nt = len(ts)
    in_specs = [pl.BlockSpec((_BM, F), lambda i: (i, 0)) for _ in range(nt)]
    in_specs.append(pl.BlockSpec((nt * F, H2), lambda i: (0, 0)))
    in_specs.append(pl.BlockSpec((8, H2), lambda i: (0, 0)))
    in_specs.append(pl.BlockSpec((1, 1, _BM), lambda i: (i, 0, 0)))
    return pl.pallas_call(
        functools.partial(_mm_pool_body, nt=nt),
        grid=(NPAD // _BM,),
        in_specs=in_specs,
        out_specs=pl.BlockSpec((NUM_GRAPHS, H2), lambda i: (0, 0)),
        out_shape=jax.ShapeDtypeStruct((NUM_GRAPHS, H2), jnp.float32),
        scratch_shapes=[
            pltpu.VMEM((NUM_GRAPHS, H2), jnp.float32),
            pltpu.VMEM((NUM_GRAPHS, 128), jnp.float32),
        ],
        interpret=_INTERPRET,
    )(*ts, wall, bias, batch3d)


def _cheb_pair(v0, dinv, u0, colp, rowp):
    txf, txb = [v0], [v0]
    uf = ub = u0
    for k in range(1, KCHEB):
        pf = _sc_matvec(uf, colp, rowp)
        pb = _sc_matvec(ub, rowp, colp)
        if k == 1:
            tf, uf = _recur(pf, v0, v0, dinv, -2.0 / 3.0, -1.0 / 3.0, 0.0)
            tb, ub = _recur(pb, v0, v0, dinv, -2.0 / 3.0, -1.0 / 3.0, 0.0)
        else:
            tf, uf = _recur(pf, txf[-1], txf[-2], dinv,
                            -4.0 / 3.0, -2.0 / 3.0, -1.0)
            tb, ub = _recur(pb, txb[-1], txb[-2], dinv,
                            -4.0 / 3.0, -2.0 / 3.0, -1.0)
        txf.append(tf)
        txb.append(tb)
    return txf, txb


def kernel(x, edge_index, batch, W1f, b1f, W1b, b1b, W2f, b2f, W2b, b2b):
    f32 = jnp.float32
    row = edge_index[0]
    col = edge_index[1]
    pad = jnp.full((EPAD - E,), TRASH, jnp.int32)
    rowp = jnp.concatenate([row, pad]).reshape(NS, NCH, CH)
    colp = jnp.concatenate([col, pad]).reshape(NS, NCH, CH)

    xp = jnp.zeros((NPAD, F), f32).at[:N].set(x)
    batch3d = jnp.zeros((NPAD,), jnp.int32).at[:N].set(batch) \
        .reshape(NPAD // _BM, 1, _BM)

    degp = _sc_degree(rowp)
    dinv, u0 = _prep(degp, xp)

    txs_f, txs_b = _cheb_pair(xp, dinv, u0, colp, rowp)
    w1 = jnp.concatenate([W1f.reshape(KCHEB * F, H1),
                          W1b.reshape(KCHEB * F, H1)], axis=0)
    bias1 = jnp.tile((b1f + b1b)[None, :], (8, 1))
    h = _mm(txs_f + txs_b, w1, bias1, H1)

    uh = _scale(dinv, h)
    txs_f2, txs_b2 = _cheb_pair(h, dinv, uh, colp, rowp)
    w2 = jnp.concatenate([W2f.reshape(KCHEB * H1, H2),
                          W2b.reshape(KCHEB * H1, H2)], axis=0)
    bias2 = jnp.tile((b2f + b2b)[None, :], (8, 1))

    return _mm_pool(txs_f2 + txs_b2, w2, bias2, batch3d)
